# baseline (speedup 1.0000x reference)
import jax
import jax.numpy as jnp
from jax import lax
from jax.experimental import pallas as pl
from jax.experimental.pallas import tpu as pltpu
from jax.experimental.pallas import tpu_sc as plsc

D = 128
DH = 64
NC = 2
NS = 16
CHUNK = 128


def _make_seg_kernel(nch, n_rows, rows_per_tile, with_deg):
    mesh = plsc.VectorSubcoreMesh(
        core_axis_name="c", subcore_axis_name="s", num_cores=NC, num_subcores=NS
    )
    K = 2
    T = nch // K
    assert nch % K == 0 and T % 2 == 0 and T >= 4
    out_type = [jax.ShapeDtypeStruct((NC, n_rows, DH), jnp.float32)]
    if with_deg:
        out_type.append(jax.ShapeDtypeStruct((NC, n_rows, 16), jnp.float32))
    scratch_types = [
        pltpu.VMEM((nch, CHUNK), jnp.int32),
        pltpu.VMEM((nch, CHUNK), jnp.int32),
        pltpu.VMEM((K * CHUNK, DH), jnp.float32),
        pltpu.VMEM((K * CHUNK, DH), jnp.float32),
        pltpu.VMEM((CHUNK, 16), jnp.float32),
        pltpu.VMEM((CHUNK, 16), jnp.float32),
        pltpu.VMEM_SHARED((n_rows, DH), jnp.float32),
        pltpu.VMEM_SHARED((n_rows, 16), jnp.float32),
        pltpu.SemaphoreType.DMA,
        pltpu.SemaphoreType.DMA,
        pltpu.SemaphoreType.DMA,
        pltpu.SemaphoreType.DMA,
        pltpu.SemaphoreType.DMA,
    ]

    def body(table, src_hbm, dst_hbm, *rest):
        if with_deg:
            (agg_out, deg_out, src_v, dst_v, buf_a, buf_b, ones_v, z16,
             agg_sh, deg_sh, ga, gb, sa, sb, so) = rest
        else:
            (agg_out, src_v, dst_v, buf_a, buf_b, ones_v, z16,
             agg_sh, deg_sh, ga, gb, sa, sb, so) = rest
        c = lax.axis_index("c")
        s = lax.axis_index("s")
        base = s * rows_per_tile
        my_tab = table.at[c]

        pltpu.sync_copy(src_hbm.at[pl.ds(s * nch, nch)], src_v)
        pltpu.sync_copy(dst_hbm.at[pl.ds(s * nch, nch)], dst_v)

        zv = jnp.zeros((16,), jnp.float32)

        @pl.loop(0, K * CHUNK * (DH // 16))
        def _(t):
            i = t // (DH // 16)
            k = t % (DH // 16)
            buf_a[i, pl.ds(k * 16, 16)] = zv

        @pl.loop(0, CHUNK)
        def _(i):
            z16[i, pl.ds(0, 16)] = zv
            ones_v[i, pl.ds(0, 16)] = zv + 1.0

        nzc = rows_per_tile // (K * CHUNK)
        for r in range(nzc):
            pltpu.sync_copy(buf_a, agg_sh.at[pl.ds(base + r * K * CHUNK, K * CHUNK)])
        rem = rows_per_tile - nzc * K * CHUNK
        if rem:
            pltpu.sync_copy(buf_a.at[pl.ds(0, rem)],
                            agg_sh.at[pl.ds(base + nzc * K * CHUNK, rem)])
        if with_deg:
            for r in range(rows_per_tile // CHUNK):
                pltpu.sync_copy(z16, deg_sh.at[pl.ds(base + r * CHUNK, CHUNK)])
        plsc.subcore_barrier()

        def issue_g(g, buf, sem):
            for k in range(K):
                pltpu.async_copy(my_tab.at[src_v.at[g * K + k]],
                                 buf.at[pl.ds(k * CHUNK, CHUNK)], sem)

        def wait_g(buf, sem):
            pltpu.make_async_copy(my_tab.at[src_v.at[0]], buf, sem).wait()

        def issue_s(g, buf, sem, parity):
            for k in range(K):
                pltpu.async_copy(buf.at[pl.ds(k * CHUNK, CHUNK)],
                                 agg_sh.at[dst_v.at[g * K + k]], sem, add=True)
            if with_deg:
                @pl.when(c == parity)
                def _():
                    for k in range(K):
                        pltpu.async_copy(ones_v, deg_sh.at[dst_v.at[g * K + k]],
                                         so, add=True)

        def wait_s(buf, sem):
            for k in range(K):
                pltpu.make_async_copy(my_tab.at[pl.ds(0, CHUNK)],
                                      buf.at[pl.ds(k * CHUNK, CHUNK)], sem).wait()

        issue_g(0, buf_a, ga)
        issue_g(1, buf_b, gb)
        wait_g(buf_a, ga)
        issue_s(0, buf_a, sa, 0)

        @pl.loop(0, (T - 2) // 2)
        def _(jj):
            j1 = 2 * jj + 1
            wait_s(buf_a, sa)
            issue_g(j1 + 1, buf_a, ga)
            wait_g(buf_b, gb)
            issue_s(j1, buf_b, sb, 1)
            wait_s(buf_b, sb)
            issue_g(j1 + 2, buf_b, gb)
            wait_g(buf_a, ga)
            issue_s(j1 + 1, buf_a, sa, 0)

        wait_s(buf_a, sa)
        wait_g(buf_b, gb)
        issue_s(T - 1, buf_b, sb, 1)
        wait_s(buf_b, sb)
        if with_deg:
            @pl.loop(0, (T // 2) * K)
            def _(t):
                pltpu.make_async_copy(deg_out.at[c, pl.ds(0, CHUNK)], ones_v,
                                      so).wait()

        plsc.subcore_barrier()

        for r in range(nzc):
            rb = base + r * K * CHUNK
            pltpu.sync_copy(agg_sh.at[pl.ds(rb, K * CHUNK)], buf_a)
            pltpu.sync_copy(buf_a, agg_out.at[c, pl.ds(rb, K * CHUNK)])
        if rem:
            rb = base + nzc * K * CHUNK
            pltpu.sync_copy(agg_sh.at[pl.ds(rb, rem)], buf_a.at[pl.ds(0, rem)])
            pltpu.sync_copy(buf_a.at[pl.ds(0, rem)], agg_out.at[c, pl.ds(rb, rem)])
        if with_deg:
            for r in range(rows_per_tile // CHUNK):
                rb = base + r * CHUNK
                pltpu.sync_copy(deg_sh.at[pl.ds(rb, CHUNK)], z16)
                pltpu.sync_copy(z16, deg_out.at[c, pl.ds(rb, CHUNK)])

    return pl.kernel(
        body, out_type=out_type, mesh=mesh, scratch_types=scratch_types,
        compiler_params=pltpu.CompilerParams(use_tc_tiling_on_sc=False),
    )


def _sage_layer1(x2, aggp, deg, w_self, w_neigh, b):
    n_rows = x2.shape[1]
    blk = 1024

    def body(x_ref, a_ref, d_ref, ws_ref, wn_ref, b_ref, o_ref):
        x = jnp.concatenate([x_ref[0], x_ref[1]], axis=1)
        agg = jnp.concatenate([a_ref[0], a_ref[1]], axis=1)
        deg = d_ref[0, :, 0:1] + d_ref[1, :, 0:1]
        mean = agg / jnp.maximum(deg, 1.0)
        h = jnp.dot(x, ws_ref[...], preferred_element_type=jnp.float32)
        h = h + jnp.dot(mean, wn_ref[...], preferred_element_type=jnp.float32)
        h = h + b_ref[...]
        h = jnp.maximum(h, 0.0)
        o_ref[0] = h[:, :DH]
        o_ref[1] = h[:, DH:]

    return pl.pallas_call(
        body,
        grid=(n_rows // blk,),
        in_specs=[
            pl.BlockSpec((NC, blk, DH), lambda i: (0, i, 0)),
            pl.BlockSpec((NC, blk, DH), lambda i: (0, i, 0)),
            pl.BlockSpec((NC, blk, 16), lambda i: (0, i, 0)),
            pl.BlockSpec((D, D), lambda i: (0, 0)),
            pl.BlockSpec((D, D), lambda i: (0, 0)),
            pl.BlockSpec((1, D), lambda i: (0, 0)),
        ],
        out_specs=pl.BlockSpec((NC, blk, DH), lambda i: (0, i, 0)),
        out_shape=jax.ShapeDtypeStruct((NC, n_rows, DH), jnp.float32),
    )(x2, aggp, deg, w_self, w_neigh, b.reshape(1, D))


def _sage_layer2(h2, aggp, deg, x2, w_self, w_neigh, b, w_res, w_mlp_pad, b_mlp_pad):
    n_rows = h2.shape[1]
    blk = 1024

    def body(h_ref, a_ref, d_ref, x_ref, ws_ref, wn_ref, b_ref, wr_ref, wm_ref, bm_ref, o_ref):
        h1 = jnp.concatenate([h_ref[0], h_ref[1]], axis=1)
        x = jnp.concatenate([x_ref[0], x_ref[1]], axis=1)
        agg = jnp.concatenate([a_ref[0], a_ref[1]], axis=1)
        deg = d_ref[0, :, 0:1] + d_ref[1, :, 0:1]
        mean = agg / jnp.maximum(deg, 1.0)
        out = jnp.dot(h1, ws_ref[...], preferred_element_type=jnp.float32)
        out = out + jnp.dot(mean, wn_ref[...], preferred_element_type=jnp.float32)
        out = out + jnp.dot(x, wr_ref[...], preferred_element_type=jnp.float32)
        out = out + b_ref[...]
        o_ref[...] = jnp.dot(out, wm_ref[...], preferred_element_type=jnp.float32) + bm_ref[...]

    return pl.pallas_call(
        body,
        grid=(n_rows // blk,),
        in_specs=[
            pl.BlockSpec((NC, blk, DH), lambda i: (0, i, 0)),
            pl.BlockSpec((NC, blk, DH), lambda i: (0, i, 0)),
            pl.BlockSpec((NC, blk, 16), lambda i: (0, i, 0)),
            pl.BlockSpec((NC, blk, DH), lambda i: (0, i, 0)),
            pl.BlockSpec((D, D), lambda i: (0, 0)),
            pl.BlockSpec((D, D), lambda i: (0, 0)),
            pl.BlockSpec((1, D), lambda i: (0, 0)),
            pl.BlockSpec((D, D), lambda i: (0, 0)),
            pl.BlockSpec((D, D), lambda i: (0, 0)),
            pl.BlockSpec((1, D), lambda i: (0, 0)),
        ],
        out_specs=pl.BlockSpec((blk, D), lambda i: (i, 0)),
        out_shape=jax.ShapeDtypeStruct((n_rows, D), jnp.float32),
    )(h2, aggp, deg, x2, w_self, w_neigh, b.reshape(1, D), w_res, w_mlp_pad, b_mlp_pad)


def kernel(node_feats, edge_index, W_self1, W_neigh1, b1, W_self2, W_neigh2, b2, W_res, W_mlp, b_mlp):
    n = node_feats.shape[0]
    e = edge_index.shape[1]
    src = edge_index[0]
    dst = edge_index[1]

    nch = -(-e // (NS * CHUNK))
    nch = -(-nch // 8) * 8
    e_pad = NS * nch * CHUNK
    src_p = jnp.concatenate([src, jnp.zeros((e_pad - e,), jnp.int32)])
    dst_p = jnp.concatenate([dst, jnp.full((e_pad - e,), n, jnp.int32)])
    src_p = src_p.reshape(e_pad // CHUNK, CHUNK)
    dst_p = dst_p.reshape(e_pad // CHUNK, CHUNK)

    rows_per_tile = -(-(n + 1) // (NS * CHUNK)) * CHUNK
    n_rows = rows_per_tile * NS

    x_pad = jnp.zeros((n_rows, D), jnp.float32).at[:n].set(node_feats)
    x2 = jnp.stack([x_pad[:, :DH], x_pad[:, DH:]])

    seg1 = _make_seg_kernel(nch, n_rows, rows_per_tile, with_deg=True)
    aggp1, deg = seg1(x2, src_p, dst_p)

    h2 = _sage_layer1(x2, aggp1, deg, W_self1, W_neigh1, b1)

    seg2 = _make_seg_kernel(nch, n_rows, rows_per_tile, with_deg=False)
    (aggp2,) = seg2(h2, src_p, dst_p)

    w_mlp_pad = jnp.zeros((D, D), jnp.float32).at[:, : W_mlp.shape[1]].set(W_mlp)
    b_mlp_pad = jnp.zeros((1, D), jnp.float32).at[0, : W_mlp.shape[1]].set(b_mlp)

    out = _sage_layer2(h2, aggp2, deg, x2, W_self2, W_neigh2, b2, W_res, w_mlp_pad, b_mlp_pad)
    return out[:n, : W_mlp.shape[1]]


# speedup vs baseline: 1.6286x; 1.6286x over previous
import jax
import jax.numpy as jnp
from jax import lax
from jax.experimental import pallas as pl
from jax.experimental.pallas import tpu as pltpu
from jax.experimental.pallas import tpu_sc as plsc

D = 128
DH = 64
NC = 2
NS = 16
CHUNK = 128


def _make_seg_kernel(nch, n_rows, rows_per_tile, with_deg):
    mesh = plsc.VectorSubcoreMesh(
        core_axis_name="c", subcore_axis_name="s", num_cores=NC, num_subcores=NS
    )
    BLK = 16
    NBLK = nch // BLK
    assert nch % BLK == 0 and nch % 2 == 0 and NBLK >= 4
    out_type = [jax.ShapeDtypeStruct((NC, n_rows, DH), jnp.float32)]
    if with_deg:
        out_type.append(jax.ShapeDtypeStruct((NC, n_rows, 16), jnp.float32))
    scratch_types = [
        pltpu.VMEM((3, BLK, CHUNK), jnp.int32),
        pltpu.VMEM((3, BLK, CHUNK), jnp.int32),
        pltpu.VMEM((CHUNK, DH), jnp.float32),
        pltpu.VMEM((CHUNK, DH), jnp.float32),
        pltpu.VMEM((CHUNK, 16), jnp.float32),
        pltpu.VMEM((CHUNK, 16), jnp.float32),
        pltpu.VMEM_SHARED((n_rows, DH), jnp.float32),
        pltpu.VMEM_SHARED((n_rows, DH), jnp.float32),
        pltpu.VMEM_SHARED((n_rows, 16), jnp.float32),
        pltpu.SemaphoreType.DMA,
        pltpu.SemaphoreType.DMA,
        pltpu.SemaphoreType.DMA,
        pltpu.SemaphoreType.DMA,
        pltpu.SemaphoreType.DMA,
        pltpu.SemaphoreType.DMA,
    ]

    def body(table, src_hbm, dst_hbm, *rest):
        if with_deg:
            (agg_out, deg_out, src_v, dst_v, buf_a, buf_b, ones_v, z16,
             tab_sh, agg_sh, deg_sh, gi, ga, gb, sa, sb, so) = rest
        else:
            (agg_out, src_v, dst_v, buf_a, buf_b, ones_v, z16,
             tab_sh, agg_sh, deg_sh, gi, ga, gb, sa, sb, so) = rest
        c = lax.axis_index("c")
        s = lax.axis_index("s")
        base = s * rows_per_tile

        def prefetch_blk(b):
            pltpu.async_copy(src_hbm.at[pl.ds(s * nch + b * BLK, BLK)],
                             src_v.at[b % 3], gi)
            pltpu.async_copy(dst_hbm.at[pl.ds(s * nch + b * BLK, BLK)],
                             dst_v.at[b % 3], gi)

        def wait_blk():
            pltpu.make_async_copy(src_hbm.at[pl.ds(0, BLK)], src_v.at[0], gi).wait()
            pltpu.make_async_copy(dst_hbm.at[pl.ds(0, BLK)], dst_v.at[0], gi).wait()

        prefetch_blk(0)

        pltpu.sync_copy(table.at[c, pl.ds(base, rows_per_tile)],
                        tab_sh.at[pl.ds(base, rows_per_tile)])

        zv = jnp.zeros((16,), jnp.float32)

        @pl.loop(0, CHUNK * (DH // 16))
        def _(t):
            i = t // (DH // 16)
            k = t % (DH // 16)
            buf_a[i, pl.ds(k * 16, 16)] = zv

        @pl.loop(0, CHUNK)
        def _(i):
            z16[i, pl.ds(0, 16)] = zv
            ones_v[i, pl.ds(0, 16)] = zv + 1.0

        for r in range(rows_per_tile // CHUNK):
            pltpu.sync_copy(buf_a, agg_sh.at[pl.ds(base + r * CHUNK, CHUNK)])
            if with_deg:
                pltpu.sync_copy(z16, deg_sh.at[pl.ds(base + r * CHUNK, CHUNK)])
        plsc.subcore_barrier()

        def src_row(j):
            return src_v.at[(j // BLK) % 3, j % BLK]

        def dst_row(j):
            return dst_v.at[(j // BLK) % 3, j % BLK]

        def issue_g(j, buf, sem):
            pltpu.async_copy(tab_sh.at[src_row(j)], buf, sem)

        def wait_g(buf, sem):
            pltpu.make_async_copy(tab_sh.at[src_row(0)], buf, sem).wait()

        def issue_s(j, buf, sem, parity):
            pltpu.async_copy(buf, agg_sh.at[dst_row(j)], sem, add=True)
            if with_deg:
                @pl.when(c == parity)
                def _():
                    pltpu.sync_copy(ones_v, deg_sh.at[dst_row(j)], add=True)

        def wait_s(buf, sem):
            pltpu.make_async_copy(table.at[c, pl.ds(0, CHUNK)], buf, sem).wait()

        def crossing(j):
            @pl.when(j % BLK == BLK - 1)
            def _():
                wait_blk()

                @pl.when(j < nch - 2 * BLK)
                def _():
                    prefetch_blk((j + 1) // BLK + 1)

        wait_blk()
        prefetch_blk(1)
        issue_g(0, buf_a, ga)
        issue_g(1, buf_b, gb)
        wait_g(buf_a, ga)
        issue_s(0, buf_a, sa, 0)

        @pl.loop(0, (nch - 2) // 2)
        def _(jj):
            j1 = 2 * jj + 1
            crossing(j1)
            wait_s(buf_a, sa)
            issue_g(j1 + 1, buf_a, ga)
            wait_g(buf_b, gb)
            issue_s(j1, buf_b, sb, 1)
            wait_s(buf_b, sb)
            issue_g(j1 + 2, buf_b, gb)
            wait_g(buf_a, ga)
            issue_s(j1 + 1, buf_a, sa, 0)

        wait_s(buf_a, sa)
        wait_g(buf_b, gb)
        issue_s(nch - 1, buf_b, sb, 1)
        wait_s(buf_b, sb)

        plsc.subcore_barrier()

        for r in range(rows_per_tile // CHUNK):
            rb = base + r * CHUNK
            pltpu.sync_copy(agg_sh.at[pl.ds(rb, CHUNK)], buf_a)
            pltpu.sync_copy(buf_a, agg_out.at[c, pl.ds(rb, CHUNK)])
            if with_deg:
                pltpu.sync_copy(deg_sh.at[pl.ds(rb, CHUNK)], z16)
                pltpu.sync_copy(z16, deg_out.at[c, pl.ds(rb, CHUNK)])

    return pl.kernel(
        body, out_type=out_type, mesh=mesh, scratch_types=scratch_types,
        compiler_params=pltpu.CompilerParams(use_tc_tiling_on_sc=False),
    )


def _sage_layer1(x2, aggp, deg, w_self, w_neigh, b):
    n_rows = x2.shape[1]
    blk = 1024

    def body(x_ref, a_ref, d_ref, ws_ref, wn_ref, b_ref, o_ref):
        x = jnp.concatenate([x_ref[0], x_ref[1]], axis=1)
        agg = jnp.concatenate([a_ref[0], a_ref[1]], axis=1)
        deg = d_ref[0, :, 0:1] + d_ref[1, :, 0:1]
        mean = agg / jnp.maximum(deg, 1.0)
        h = jnp.dot(x, ws_ref[...], preferred_element_type=jnp.float32)
        h = h + jnp.dot(mean, wn_ref[...], preferred_element_type=jnp.float32)
        h = h + b_ref[...]
        h = jnp.maximum(h, 0.0)
        o_ref[0] = h[:, :DH]
        o_ref[1] = h[:, DH:]

    return pl.pallas_call(
        body,
        grid=(n_rows // blk,),
        in_specs=[
            pl.BlockSpec((NC, blk, DH), lambda i: (0, i, 0)),
            pl.BlockSpec((NC, blk, DH), lambda i: (0, i, 0)),
            pl.BlockSpec((NC, blk, 16), lambda i: (0, i, 0)),
            pl.BlockSpec((D, D), lambda i: (0, 0)),
            pl.BlockSpec((D, D), lambda i: (0, 0)),
            pl.BlockSpec((1, D), lambda i: (0, 0)),
        ],
        out_specs=pl.BlockSpec((NC, blk, DH), lambda i: (0, i, 0)),
        out_shape=jax.ShapeDtypeStruct((NC, n_rows, DH), jnp.float32),
    )(x2, aggp, deg, w_self, w_neigh, b.reshape(1, D))


def _sage_layer2(h2, aggp, deg, x2, w_self, w_neigh, b, w_res, w_mlp_pad, b_mlp_pad):
    n_rows = h2.shape[1]
    blk = 1024

    def body(h_ref, a_ref, d_ref, x_ref, ws_ref, wn_ref, b_ref, wr_ref, wm_ref, bm_ref, o_ref):
        h1 = jnp.concatenate([h_ref[0], h_ref[1]], axis=1)
        x = jnp.concatenate([x_ref[0], x_ref[1]], axis=1)
        agg = jnp.concatenate([a_ref[0], a_ref[1]], axis=1)
        deg = d_ref[0, :, 0:1] + d_ref[1, :, 0:1]
        mean = agg / jnp.maximum(deg, 1.0)
        out = jnp.dot(h1, ws_ref[...], preferred_element_type=jnp.float32)
        out = out + jnp.dot(mean, wn_ref[...], preferred_element_type=jnp.float32)
        out = out + jnp.dot(x, wr_ref[...], preferred_element_type=jnp.float32)
        out = out + b_ref[...]
        o_ref[...] = jnp.dot(out, wm_ref[...], preferred_element_type=jnp.float32) + bm_ref[...]

    return pl.pallas_call(
        body,
        grid=(n_rows // blk,),
        in_specs=[
            pl.BlockSpec((NC, blk, DH), lambda i: (0, i, 0)),
            pl.BlockSpec((NC, blk, DH), lambda i: (0, i, 0)),
            pl.BlockSpec((NC, blk, 16), lambda i: (0, i, 0)),
            pl.BlockSpec((NC, blk, DH), lambda i: (0, i, 0)),
            pl.BlockSpec((D, D), lambda i: (0, 0)),
            pl.BlockSpec((D, D), lambda i: (0, 0)),
            pl.BlockSpec((1, D), lambda i: (0, 0)),
            pl.BlockSpec((D, D), lambda i: (0, 0)),
            pl.BlockSpec((D, D), lambda i: (0, 0)),
            pl.BlockSpec((1, D), lambda i: (0, 0)),
        ],
        out_specs=pl.BlockSpec((blk, D), lambda i: (i, 0)),
        out_shape=jax.ShapeDtypeStruct((n_rows, D), jnp.float32),
    )(h2, aggp, deg, x2, w_self, w_neigh, b.reshape(1, D), w_res, w_mlp_pad, b_mlp_pad)


def kernel(node_feats, edge_index, W_self1, W_neigh1, b1, W_self2, W_neigh2, b2, W_res, W_mlp, b_mlp):
    n = node_feats.shape[0]
    e = edge_index.shape[1]
    src = edge_index[0]
    dst = edge_index[1]

    rows_per_tile = -(-(n + 1) // (NS * CHUNK)) * CHUNK
    n_rows = rows_per_tile * NS

    nch = -(-e // (NS * CHUNK))
    nch = -(-nch // 16) * 16
    e_pad = NS * nch * CHUNK
    pad_i = jnp.arange(e_pad - e, dtype=jnp.int32)
    src_p = jnp.concatenate([src, pad_i % n])
    dst_p = jnp.concatenate([dst, n + pad_i % (n_rows - n)])
    src_p = src_p.reshape(e_pad // CHUNK, CHUNK)
    dst_p = dst_p.reshape(e_pad // CHUNK, CHUNK)

    x_pad = jnp.zeros((n_rows, D), jnp.float32).at[:n].set(node_feats)
    x2 = jnp.stack([x_pad[:, :DH], x_pad[:, DH:]])

    seg1 = _make_seg_kernel(nch, n_rows, rows_per_tile, with_deg=True)
    aggp1, deg = seg1(x2, src_p, dst_p)

    h2 = _sage_layer1(x2, aggp1, deg, W_self1, W_neigh1, b1)

    seg2 = _make_seg_kernel(nch, n_rows, rows_per_tile, with_deg=False)
    (aggp2,) = seg2(h2, src_p, dst_p)

    w_mlp_pad = jnp.zeros((D, D), jnp.float32).at[:, : W_mlp.shape[1]].set(W_mlp)
    b_mlp_pad = jnp.zeros((1, D), jnp.float32).at[0, : W_mlp.shape[1]].set(b_mlp)

    out = _sage_layer2(h2, aggp2, deg, x2, W_self2, W_neigh2, b2, W_res, w_mlp_pad, b_mlp_pad)
    return out[:n, : W_mlp.shape[1]]


# speedup vs baseline: 1.6534x; 1.0152x over previous
import jax
import jax.numpy as jnp
from jax import lax
from jax.experimental import pallas as pl
from jax.experimental.pallas import tpu as pltpu
from jax.experimental.pallas import tpu_sc as plsc

D = 128
DH = 64
NC = 2
NS = 16
CHUNK = 128


def _make_seg_kernel(nch, n_rows, rows_per_tile, with_deg):
    mesh = plsc.VectorSubcoreMesh(
        core_axis_name="c", subcore_axis_name="s", num_cores=NC, num_subcores=NS
    )
    BLK = 16
    NBLK = nch // BLK
    assert nch % BLK == 0 and nch % 2 == 0 and NBLK >= 4
    out_type = [jax.ShapeDtypeStruct((NC, n_rows, DH), jnp.float32)]
    if with_deg:
        out_type.append(jax.ShapeDtypeStruct((NC, n_rows, 16), jnp.float32))
    scratch_types = [
        pltpu.VMEM((3, BLK, CHUNK), jnp.int32),
        pltpu.VMEM((3, BLK, CHUNK), jnp.int32),
        pltpu.VMEM((CHUNK, DH), jnp.float32),
        pltpu.VMEM((CHUNK, DH), jnp.float32),
        pltpu.VMEM((CHUNK, 16), jnp.float32),
        pltpu.VMEM((CHUNK, 16), jnp.float32),
        pltpu.VMEM_SHARED((n_rows, DH), jnp.float32),
        pltpu.VMEM_SHARED((n_rows, DH), jnp.float32),
        pltpu.VMEM_SHARED((n_rows, 16), jnp.float32),
        pltpu.SemaphoreType.DMA,
        pltpu.SemaphoreType.DMA,
        pltpu.SemaphoreType.DMA,
        pltpu.SemaphoreType.DMA,
        pltpu.SemaphoreType.DMA,
        pltpu.SemaphoreType.DMA,
    ]

    def body(table, src_hbm, dst_hbm, *rest):
        if with_deg:
            (agg_out, deg_out, src_v, dst_v, buf_a, buf_b, ones_v, z16,
             tab_sh, agg_sh, deg_sh, gi, ga, gb, sa, sb, so) = rest
        else:
            (agg_out, src_v, dst_v, buf_a, buf_b, ones_v, z16,
             tab_sh, agg_sh, deg_sh, gi, ga, gb, sa, sb, so) = rest
        c = lax.axis_index("c")
        s = lax.axis_index("s")
        base = s * rows_per_tile

        def prefetch_blk(b):
            pltpu.async_copy(src_hbm.at[pl.ds(s * nch + b * BLK, BLK)],
                             src_v.at[b % 3], gi)
            pltpu.async_copy(dst_hbm.at[pl.ds(s * nch + b * BLK, BLK)],
                             dst_v.at[b % 3], gi)

        def wait_blk():
            pltpu.make_async_copy(src_hbm.at[pl.ds(0, BLK)], src_v.at[0], gi).wait()
            pltpu.make_async_copy(dst_hbm.at[pl.ds(0, BLK)], dst_v.at[0], gi).wait()

        prefetch_blk(0)

        pltpu.sync_copy(table.at[c, pl.ds(base, rows_per_tile)],
                        tab_sh.at[pl.ds(base, rows_per_tile)])

        zv = jnp.zeros((16,), jnp.float32)

        @pl.loop(0, CHUNK * (DH // 16))
        def _(t):
            i = t // (DH // 16)
            k = t % (DH // 16)
            buf_a[i, pl.ds(k * 16, 16)] = zv

        @pl.loop(0, CHUNK)
        def _(i):
            z16[i, pl.ds(0, 16)] = zv
            ones_v[i, pl.ds(0, 16)] = zv + 1.0

        for r in range(rows_per_tile // CHUNK):
            pltpu.sync_copy(buf_a, agg_sh.at[pl.ds(base + r * CHUNK, CHUNK)])
            if with_deg:
                pltpu.sync_copy(z16, deg_sh.at[pl.ds(base + r * CHUNK, CHUNK)])
        plsc.subcore_barrier()

        def src_row(j):
            return src_v.at[(j // BLK) % 3, j % BLK]

        def dst_row(j):
            return dst_v.at[(j // BLK) % 3, j % BLK]

        def issue_g(j, buf, sem):
            pltpu.async_copy(tab_sh.at[src_row(j)], buf, sem)

        def wait_g(buf, sem):
            pltpu.make_async_copy(tab_sh.at[src_row(0)], buf, sem).wait()

        def issue_s(j, buf, sem, parity):
            pltpu.async_copy(buf, agg_sh.at[dst_row(j)], sem, add=True)
            if with_deg:
                @pl.when(c == parity)
                def _():
                    pltpu.sync_copy(ones_v, deg_sh.at[dst_row(j)], add=True)

        def wait_s(buf, sem):
            pltpu.make_async_copy(table.at[c, pl.ds(0, CHUNK)], buf, sem).wait()

        def crossing(j):
            @pl.when(j % BLK == BLK - 1)
            def _():
                wait_blk()

                @pl.when(j < nch - 2 * BLK)
                def _():
                    prefetch_blk((j + 1) // BLK + 1)

        wait_blk()
        prefetch_blk(1)
        issue_g(0, buf_a, ga)
        issue_g(1, buf_b, gb)
        wait_g(buf_a, ga)
        issue_s(0, buf_a, sa, 0)

        @pl.loop(0, (nch - 2) // 2)
        def _(jj):
            j1 = 2 * jj + 1
            crossing(j1)
            wait_s(buf_a, sa)
            issue_g(j1 + 1, buf_a, ga)
            wait_g(buf_b, gb)
            issue_s(j1, buf_b, sb, 1)
            wait_s(buf_b, sb)
            issue_g(j1 + 2, buf_b, gb)
            wait_g(buf_a, ga)
            issue_s(j1 + 1, buf_a, sa, 0)

        wait_s(buf_a, sa)
        wait_g(buf_b, gb)
        issue_s(nch - 1, buf_b, sb, 1)
        wait_s(buf_b, sb)

        plsc.subcore_barrier()

        for r in range(rows_per_tile // CHUNK):
            rb = base + r * CHUNK
            pltpu.sync_copy(agg_sh.at[pl.ds(rb, CHUNK)], buf_a)
            pltpu.sync_copy(buf_a, agg_out.at[c, pl.ds(rb, CHUNK)])
            if with_deg:
                pltpu.sync_copy(deg_sh.at[pl.ds(rb, CHUNK)], z16)
                pltpu.sync_copy(z16, deg_out.at[c, pl.ds(rb, CHUNK)])

    return pl.kernel(
        body, out_type=out_type, mesh=mesh, scratch_types=scratch_types,
        compiler_params=pltpu.CompilerParams(use_tc_tiling_on_sc=False),
    )


def _pre1(x2, w_self, b, w_res):
    n_rows = x2.shape[1]
    blk = 1024

    def body(x_ref, ws_ref, b_ref, wr_ref, t_ref, r_ref):
        x = jnp.concatenate([x_ref[0], x_ref[1]], axis=1)
        t_ref[...] = jnp.dot(x, ws_ref[...], preferred_element_type=jnp.float32) + b_ref[...]
        r_ref[...] = jnp.dot(x, wr_ref[...], preferred_element_type=jnp.float32)

    return pl.pallas_call(
        body,
        grid=(n_rows // blk,),
        in_specs=[
            pl.BlockSpec((NC, blk, DH), lambda i: (0, i, 0)),
            pl.BlockSpec((D, D), lambda i: (0, 0)),
            pl.BlockSpec((1, D), lambda i: (0, 0)),
            pl.BlockSpec((D, D), lambda i: (0, 0)),
        ],
        out_specs=[
            pl.BlockSpec((blk, D), lambda i: (i, 0)),
            pl.BlockSpec((blk, D), lambda i: (i, 0)),
        ],
        out_shape=[
            jax.ShapeDtypeStruct((n_rows, D), jnp.float32),
            jax.ShapeDtypeStruct((n_rows, D), jnp.float32),
        ],
    )(x2, w_self, b.reshape(1, D), w_res)


def _post1(t1, aggp, deg, w_neigh):
    n_rows = t1.shape[0]
    blk = 1024

    def body(t_ref, a_ref, d_ref, wn_ref, o_ref):
        agg = jnp.concatenate([a_ref[0], a_ref[1]], axis=1)
        dg = d_ref[0, :, 0:1] + d_ref[1, :, 0:1]
        mean = agg / jnp.maximum(dg, 1.0)
        h = t_ref[...] + jnp.dot(mean, wn_ref[...], preferred_element_type=jnp.float32)
        h = jnp.maximum(h, 0.0)
        o_ref[0] = h[:, :DH]
        o_ref[1] = h[:, DH:]

    return pl.pallas_call(
        body,
        grid=(n_rows // blk,),
        in_specs=[
            pl.BlockSpec((blk, D), lambda i: (i, 0)),
            pl.BlockSpec((NC, blk, DH), lambda i: (0, i, 0)),
            pl.BlockSpec((NC, blk, 16), lambda i: (0, i, 0)),
            pl.BlockSpec((D, D), lambda i: (0, 0)),
        ],
        out_specs=pl.BlockSpec((NC, blk, DH), lambda i: (0, i, 0)),
        out_shape=jax.ShapeDtypeStruct((NC, n_rows, DH), jnp.float32),
    )(t1, aggp, deg, w_neigh)


def _pre2(h2, r, w_self, b):
    n_rows = h2.shape[1]
    blk = 1024

    def body(h_ref, r_ref, ws_ref, b_ref, o_ref):
        h1 = jnp.concatenate([h_ref[0], h_ref[1]], axis=1)
        o_ref[...] = (jnp.dot(h1, ws_ref[...], preferred_element_type=jnp.float32)
                      + b_ref[...] + r_ref[...])

    return pl.pallas_call(
        body,
        grid=(n_rows // blk,),
        in_specs=[
            pl.BlockSpec((NC, blk, DH), lambda i: (0, i, 0)),
            pl.BlockSpec((blk, D), lambda i: (i, 0)),
            pl.BlockSpec((D, D), lambda i: (0, 0)),
            pl.BlockSpec((1, D), lambda i: (0, 0)),
        ],
        out_specs=pl.BlockSpec((blk, D), lambda i: (i, 0)),
        out_shape=jax.ShapeDtypeStruct((n_rows, D), jnp.float32),
    )(h2, r, w_self, b.reshape(1, D))


def _post2(t2, aggp, deg, w_neigh, w_mlp_pad, b_mlp_pad):
    n_rows = t2.shape[0]
    blk = 1024

    def body(t_ref, a_ref, d_ref, wn_ref, wm_ref, bm_ref, o_ref):
        agg = jnp.concatenate([a_ref[0], a_ref[1]], axis=1)
        dg = d_ref[0, :, 0:1] + d_ref[1, :, 0:1]
        mean = agg / jnp.maximum(dg, 1.0)
        h = t_ref[...] + jnp.dot(mean, wn_ref[...], preferred_element_type=jnp.float32)
        o_ref[...] = jnp.dot(h, wm_ref[...], preferred_element_type=jnp.float32) + bm_ref[...]

    return pl.pallas_call(
        body,
        grid=(n_rows // blk,),
        in_specs=[
            pl.BlockSpec((blk, D), lambda i: (i, 0)),
            pl.BlockSpec((NC, blk, DH), lambda i: (0, i, 0)),
            pl.BlockSpec((NC, blk, 16), lambda i: (0, i, 0)),
            pl.BlockSpec((D, D), lambda i: (0, 0)),
            pl.BlockSpec((D, D), lambda i: (0, 0)),
            pl.BlockSpec((1, D), lambda i: (0, 0)),
        ],
        out_specs=pl.BlockSpec((blk, D), lambda i: (i, 0)),
        out_shape=jax.ShapeDtypeStruct((n_rows, D), jnp.float32),
    )(t2, aggp, deg, w_neigh, w_mlp_pad, b_mlp_pad)


def kernel(node_feats, edge_index, W_self1, W_neigh1, b1, W_self2, W_neigh2, b2, W_res, W_mlp, b_mlp):
    n = node_feats.shape[0]
    e = edge_index.shape[1]
    src = edge_index[0]
    dst = edge_index[1]

    rows_per_tile = -(-(n + 1) // (NS * CHUNK)) * CHUNK
    n_rows = rows_per_tile * NS

    nch = -(-e // (NS * CHUNK))
    nch = -(-nch // 16) * 16
    e_pad = NS * nch * CHUNK
    pad_i = jnp.arange(e_pad - e, dtype=jnp.int32)
    src_p = jnp.concatenate([src, pad_i % n])
    dst_p = jnp.concatenate([dst, n + pad_i % (n_rows - n)])
    src_p = src_p.reshape(e_pad // CHUNK, CHUNK)
    dst_p = dst_p.reshape(e_pad // CHUNK, CHUNK)

    x_pad = jnp.zeros((n_rows, D), jnp.float32).at[:n].set(node_feats)
    x2 = jnp.stack([x_pad[:, :DH], x_pad[:, DH:]])

    seg1 = _make_seg_kernel(nch, n_rows, rows_per_tile, with_deg=True)
    aggp1, deg = seg1(x2, src_p, dst_p)
    t1, r = _pre1(x2, W_self1, b1, W_res)

    h2 = _post1(t1, aggp1, deg, W_neigh1)

    seg2 = _make_seg_kernel(nch, n_rows, rows_per_tile, with_deg=False)
    (aggp2,) = seg2(h2, src_p, dst_p)
    t2 = _pre2(h2, r, W_self2, b2)

    w_mlp_pad = jnp.zeros((D, D), jnp.float32).at[:, : W_mlp.shape[1]].set(W_mlp)
    b_mlp_pad = jnp.zeros((1, D), jnp.float32).at[0, : W_mlp.shape[1]].set(b_mlp)

    out = _post2(t2, aggp2, deg, W_neigh2, w_mlp_pad, b_mlp_pad)
    return out[:n, : W_mlp.shape[1]]


# speedup vs baseline: 1.6814x; 1.0170x over previous
import jax
import jax.numpy as jnp
from jax import lax
from jax.experimental import pallas as pl
from jax.experimental.pallas import tpu as pltpu
from jax.experimental.pallas import tpu_sc as plsc

D = 128
DH = 64
NC = 2
NS = 16
CHUNK = 128


def _make_seg_kernel(nch, n_rows, rows_per_tile, with_deg):
    mesh = plsc.VectorSubcoreMesh(
        core_axis_name="c", subcore_axis_name="s", num_cores=NC, num_subcores=NS
    )
    BLK = 16
    NBLK = nch // BLK
    assert nch % BLK == 0 and nch % 2 == 0 and NBLK >= 4
    out_type = [jax.ShapeDtypeStruct((NC, n_rows, DH), jnp.float32)]
    if with_deg:
        out_type.append(jax.ShapeDtypeStruct((NC, n_rows, 16), jnp.float32))
    scratch_types = [
        pltpu.VMEM((3, BLK, CHUNK), jnp.int32),
        pltpu.VMEM((3, BLK, CHUNK), jnp.int32),
        pltpu.VMEM((CHUNK, DH), jnp.float32),
        pltpu.VMEM((CHUNK, DH), jnp.float32),
        pltpu.VMEM((CHUNK, 16), jnp.float32),
        pltpu.VMEM((CHUNK, 16), jnp.float32),
        pltpu.VMEM_SHARED((n_rows, DH), jnp.float32),
        pltpu.VMEM_SHARED((n_rows, DH), jnp.float32),
        pltpu.VMEM_SHARED((n_rows, 16), jnp.float32),
        pltpu.SemaphoreType.DMA,
        pltpu.SemaphoreType.DMA,
        pltpu.SemaphoreType.DMA,
        pltpu.SemaphoreType.DMA,
        pltpu.SemaphoreType.DMA,
        pltpu.SemaphoreType.DMA,
    ]

    def body(table, src_hbm, dst_hbm, *rest):
        if with_deg:
            (agg_out, deg_out, src_v, dst_v, buf_a, buf_b, ones_v, z16,
             tab_sh, agg_sh, deg_sh, gi, ga, gb, sa, sb, so) = rest
        else:
            (agg_out, src_v, dst_v, buf_a, buf_b, ones_v, z16,
             tab_sh, agg_sh, deg_sh, gi, ga, gb, sa, sb, so) = rest
        c = lax.axis_index("c")
        s = lax.axis_index("s")
        base = s * rows_per_tile

        def prefetch_blk(b):
            pltpu.async_copy(src_hbm.at[pl.ds(s * nch + b * BLK, BLK)],
                             src_v.at[b % 3], gi)
            pltpu.async_copy(dst_hbm.at[pl.ds(s * nch + b * BLK, BLK)],
                             dst_v.at[b % 3], gi)

        def wait_blk():
            pltpu.make_async_copy(src_hbm.at[pl.ds(0, BLK)], src_v.at[0], gi).wait()
            pltpu.make_async_copy(dst_hbm.at[pl.ds(0, BLK)], dst_v.at[0], gi).wait()

        prefetch_blk(0)

        pltpu.sync_copy(table.at[c, pl.ds(base, rows_per_tile)],
                        tab_sh.at[pl.ds(base, rows_per_tile)])

        zv = jnp.zeros((16,), jnp.float32)

        @pl.loop(0, CHUNK * (DH // 16))
        def _(t):
            i = t // (DH // 16)
            k = t % (DH // 16)
            buf_a[i, pl.ds(k * 16, 16)] = zv

        @pl.loop(0, CHUNK)
        def _(i):
            z16[i, pl.ds(0, 16)] = zv
            ones_v[i, pl.ds(0, 16)] = zv + 1.0

        for r in range(rows_per_tile // CHUNK):
            pltpu.sync_copy(buf_a, agg_sh.at[pl.ds(base + r * CHUNK, CHUNK)])
            if with_deg:
                pltpu.sync_copy(z16, deg_sh.at[pl.ds(base + r * CHUNK, CHUNK)])
        plsc.subcore_barrier()

        def src_row(j):
            return src_v.at[(j // BLK) % 3, j % BLK]

        def dst_row(j):
            return dst_v.at[(j // BLK) % 3, j % BLK]

        def issue_g(j, buf, sem):
            pltpu.async_copy(tab_sh.at[src_row(j)], buf, sem)

        def wait_g(buf, sem):
            pltpu.make_async_copy(tab_sh.at[src_row(0)], buf, sem).wait()

        def issue_s(j, buf, sem, parity):
            pltpu.async_copy(buf, agg_sh.at[dst_row(j)], sem, add=True)

        def drain_ones():
            pltpu.make_async_copy(deg_out.at[c, pl.ds(0, CHUNK)], ones_v,
                                  so).wait()

        def issue_ones(j, parity, drain):
            if not with_deg:
                return
            @pl.when(c == parity)
            def _():
                if drain:
                    drain_ones()
                pltpu.async_copy(ones_v, deg_sh.at[dst_row(j)], so, add=True)

        def wait_s(buf, sem):
            pltpu.make_async_copy(table.at[c, pl.ds(0, CHUNK)], buf, sem).wait()

        def crossing(j):
            @pl.when(j % BLK == BLK - 1)
            def _():
                wait_blk()

                @pl.when(j < nch - 2 * BLK)
                def _():
                    prefetch_blk((j + 1) // BLK + 1)

        wait_blk()
        prefetch_blk(1)
        issue_g(0, buf_a, ga)
        issue_g(1, buf_b, gb)
        wait_g(buf_a, ga)
        issue_s(0, buf_a, sa, 0)
        issue_ones(0, 0, drain=False)

        @pl.loop(0, (nch - 2) // 2)
        def _(jj):
            j1 = 2 * jj + 1
            crossing(j1)
            wait_s(buf_a, sa)
            issue_g(j1 + 1, buf_a, ga)
            wait_g(buf_b, gb)
            issue_s(j1, buf_b, sb, 1)
            if with_deg:
                @pl.when(jnp.logical_and(c == 1, jj > 0))
                def _():
                    drain_ones()
                @pl.when(c == 1)
                def _():
                    pltpu.async_copy(ones_v, deg_sh.at[dst_row(j1)], so, add=True)
            wait_s(buf_b, sb)
            issue_g(j1 + 2, buf_b, gb)
            wait_g(buf_a, ga)
            issue_s(j1 + 1, buf_a, sa, 0)
            issue_ones(j1 + 1, 0, drain=True)

        wait_s(buf_a, sa)
        wait_g(buf_b, gb)
        issue_s(nch - 1, buf_b, sb, 1)
        issue_ones(nch - 1, 1, drain=True)
        wait_s(buf_b, sb)
        if with_deg:
            drain_ones()

        plsc.subcore_barrier()

        for r in range(rows_per_tile // CHUNK):
            rb = base + r * CHUNK
            pltpu.sync_copy(agg_sh.at[pl.ds(rb, CHUNK)], buf_a)
            pltpu.sync_copy(buf_a, agg_out.at[c, pl.ds(rb, CHUNK)])
            if with_deg:
                pltpu.sync_copy(deg_sh.at[pl.ds(rb, CHUNK)], z16)
                pltpu.sync_copy(z16, deg_out.at[c, pl.ds(rb, CHUNK)])

    return pl.kernel(
        body, out_type=out_type, mesh=mesh, scratch_types=scratch_types,
        compiler_params=pltpu.CompilerParams(use_tc_tiling_on_sc=False),
    )


def _pre1(x2, w_self, b, w_res):
    n_rows = x2.shape[1]
    blk = 1024

    def body(x_ref, ws_ref, b_ref, wr_ref, t_ref, r_ref):
        x = jnp.concatenate([x_ref[0], x_ref[1]], axis=1)
        t_ref[...] = jnp.dot(x, ws_ref[...], preferred_element_type=jnp.float32) + b_ref[...]
        r_ref[...] = jnp.dot(x, wr_ref[...], preferred_element_type=jnp.float32)

    return pl.pallas_call(
        body,
        grid=(n_rows // blk,),
        in_specs=[
            pl.BlockSpec((NC, blk, DH), lambda i: (0, i, 0)),
            pl.BlockSpec((D, D), lambda i: (0, 0)),
            pl.BlockSpec((1, D), lambda i: (0, 0)),
            pl.BlockSpec((D, D), lambda i: (0, 0)),
        ],
        out_specs=[
            pl.BlockSpec((blk, D), lambda i: (i, 0)),
            pl.BlockSpec((blk, D), lambda i: (i, 0)),
        ],
        out_shape=[
            jax.ShapeDtypeStruct((n_rows, D), jnp.float32),
            jax.ShapeDtypeStruct((n_rows, D), jnp.float32),
        ],
    )(x2, w_self, b.reshape(1, D), w_res)


def _post1(t1, aggp, deg, w_neigh):
    n_rows = t1.shape[0]
    blk = 1024

    def body(t_ref, a_ref, d_ref, wn_ref, o_ref):
        agg = jnp.concatenate([a_ref[0], a_ref[1]], axis=1)
        dg = d_ref[0, :, 0:1] + d_ref[1, :, 0:1]
        mean = agg / jnp.maximum(dg, 1.0)
        h = t_ref[...] + jnp.dot(mean, wn_ref[...], preferred_element_type=jnp.float32)
        h = jnp.maximum(h, 0.0)
        o_ref[0] = h[:, :DH]
        o_ref[1] = h[:, DH:]

    return pl.pallas_call(
        body,
        grid=(n_rows // blk,),
        in_specs=[
            pl.BlockSpec((blk, D), lambda i: (i, 0)),
            pl.BlockSpec((NC, blk, DH), lambda i: (0, i, 0)),
            pl.BlockSpec((NC, blk, 16), lambda i: (0, i, 0)),
            pl.BlockSpec((D, D), lambda i: (0, 0)),
        ],
        out_specs=pl.BlockSpec((NC, blk, DH), lambda i: (0, i, 0)),
        out_shape=jax.ShapeDtypeStruct((NC, n_rows, DH), jnp.float32),
    )(t1, aggp, deg, w_neigh)


def _pre2(h2, r, w_self, b):
    n_rows = h2.shape[1]
    blk = 1024

    def body(h_ref, r_ref, ws_ref, b_ref, o_ref):
        h1 = jnp.concatenate([h_ref[0], h_ref[1]], axis=1)
        o_ref[...] = (jnp.dot(h1, ws_ref[...], preferred_element_type=jnp.float32)
                      + b_ref[...] + r_ref[...])

    return pl.pallas_call(
        body,
        grid=(n_rows // blk,),
        in_specs=[
            pl.BlockSpec((NC, blk, DH), lambda i: (0, i, 0)),
            pl.BlockSpec((blk, D), lambda i: (i, 0)),
            pl.BlockSpec((D, D), lambda i: (0, 0)),
            pl.BlockSpec((1, D), lambda i: (0, 0)),
        ],
        out_specs=pl.BlockSpec((blk, D), lambda i: (i, 0)),
        out_shape=jax.ShapeDtypeStruct((n_rows, D), jnp.float32),
    )(h2, r, w_self, b.reshape(1, D))


def _post2(t2, aggp, deg, w_neigh, w_mlp_pad, b_mlp_pad):
    n_rows = t2.shape[0]
    blk = 1024

    def body(t_ref, a_ref, d_ref, wn_ref, wm_ref, bm_ref, o_ref):
        agg = jnp.concatenate([a_ref[0], a_ref[1]], axis=1)
        dg = d_ref[0, :, 0:1] + d_ref[1, :, 0:1]
        mean = agg / jnp.maximum(dg, 1.0)
        h = t_ref[...] + jnp.dot(mean, wn_ref[...], preferred_element_type=jnp.float32)
        o_ref[...] = jnp.dot(h, wm_ref[...], preferred_element_type=jnp.float32) + bm_ref[...]

    return pl.pallas_call(
        body,
        grid=(n_rows // blk,),
        in_specs=[
            pl.BlockSpec((blk, D), lambda i: (i, 0)),
            pl.BlockSpec((NC, blk, DH), lambda i: (0, i, 0)),
            pl.BlockSpec((NC, blk, 16), lambda i: (0, i, 0)),
            pl.BlockSpec((D, D), lambda i: (0, 0)),
            pl.BlockSpec((D, D), lambda i: (0, 0)),
            pl.BlockSpec((1, D), lambda i: (0, 0)),
        ],
        out_specs=pl.BlockSpec((blk, D), lambda i: (i, 0)),
        out_shape=jax.ShapeDtypeStruct((n_rows, D), jnp.float32),
    )(t2, aggp, deg, w_neigh, w_mlp_pad, b_mlp_pad)


def kernel(node_feats, edge_index, W_self1, W_neigh1, b1, W_self2, W_neigh2, b2, W_res, W_mlp, b_mlp):
    n = node_feats.shape[0]
    e = edge_index.shape[1]
    src = edge_index[0]
    dst = edge_index[1]

    rows_per_tile = -(-(n + 1) // (NS * CHUNK)) * CHUNK
    n_rows = rows_per_tile * NS

    nch = -(-e // (NS * CHUNK))
    nch = -(-nch // 16) * 16
    e_pad = NS * nch * CHUNK
    pad_i = jnp.arange(e_pad - e, dtype=jnp.int32)
    src_p = jnp.concatenate([src, pad_i % n])
    dst_p = jnp.concatenate([dst, n + pad_i % (n_rows - n)])
    src_p = src_p.reshape(e_pad // CHUNK, CHUNK)
    dst_p = dst_p.reshape(e_pad // CHUNK, CHUNK)

    x_pad = jnp.zeros((n_rows, D), jnp.float32).at[:n].set(node_feats)
    x2 = jnp.stack([x_pad[:, :DH], x_pad[:, DH:]])

    seg1 = _make_seg_kernel(nch, n_rows, rows_per_tile, with_deg=True)
    aggp1, deg = seg1(x2, src_p, dst_p)
    t1, r = _pre1(x2, W_self1, b1, W_res)

    h2 = _post1(t1, aggp1, deg, W_neigh1)

    seg2 = _make_seg_kernel(nch, n_rows, rows_per_tile, with_deg=False)
    (aggp2,) = seg2(h2, src_p, dst_p)
    t2 = _pre2(h2, r, W_self2, b2)

    w_mlp_pad = jnp.zeros((D, D), jnp.float32).at[:, : W_mlp.shape[1]].set(W_mlp)
    b_mlp_pad = jnp.zeros((1, D), jnp.float32).at[0, : W_mlp.shape[1]].set(b_mlp)

    out = _post2(t2, aggp2, deg, W_neigh2, w_mlp_pad, b_mlp_pad)
    return out[:n, : W_mlp.shape[1]]


# speedup vs baseline: 1.6849x; 1.0021x over previous
import jax
import jax.numpy as jnp
from jax import lax
from jax.experimental import pallas as pl
from jax.experimental.pallas import tpu as pltpu
from jax.experimental.pallas import tpu_sc as plsc

D = 128
DH = 64
NC = 2
NS = 16
CHUNK = 128


def _make_seg_kernel(nch, n_rows, rows_per_tile, with_deg):
    mesh = plsc.VectorSubcoreMesh(
        core_axis_name="c", subcore_axis_name="s", num_cores=NC, num_subcores=NS
    )
    BLK = 16
    NBLK = nch // BLK
    assert nch % BLK == 0 and nch % 2 == 0 and NBLK >= 4
    out_type = [jax.ShapeDtypeStruct((NC, n_rows, DH), jnp.float32)]
    if with_deg:
        out_type.append(jax.ShapeDtypeStruct((NC, n_rows, 16), jnp.float32))
    scratch_types = [
        pltpu.VMEM((3, BLK, CHUNK), jnp.int32),
        pltpu.VMEM((3, BLK, CHUNK), jnp.int32),
        pltpu.VMEM((CHUNK, DH), jnp.float32),
        pltpu.VMEM((CHUNK, DH), jnp.float32),
        pltpu.VMEM((CHUNK, 16), jnp.float32),
        pltpu.VMEM((CHUNK, 16), jnp.float32),
        pltpu.VMEM_SHARED((n_rows, DH), jnp.float32),
        pltpu.VMEM_SHARED((n_rows, DH), jnp.float32),
        pltpu.VMEM_SHARED((n_rows, 16), jnp.float32),
        pltpu.SemaphoreType.DMA,
        pltpu.SemaphoreType.DMA,
        pltpu.SemaphoreType.DMA,
        pltpu.SemaphoreType.DMA,
        pltpu.SemaphoreType.DMA,
        pltpu.SemaphoreType.DMA,
    ]

    def body(table, src_hbm, dst_hbm, *rest):
        if with_deg:
            (agg_out, deg_out, src_v, dst_v, buf_a, buf_b, ones_v, z16,
             tab_sh, agg_sh, deg_sh, gi, ga, gb, sa, sb, so) = rest
        else:
            (agg_out, src_v, dst_v, buf_a, buf_b, ones_v, z16,
             tab_sh, agg_sh, deg_sh, gi, ga, gb, sa, sb, so) = rest
        c = lax.axis_index("c")
        s = lax.axis_index("s")
        base = s * rows_per_tile

        def prefetch_blk(b):
            pltpu.async_copy(src_hbm.at[pl.ds(s * nch + b * BLK, BLK)],
                             src_v.at[b % 3], gi)
            pltpu.async_copy(dst_hbm.at[pl.ds(s * nch + b * BLK, BLK)],
                             dst_v.at[b % 3], gi)

        def wait_blk():
            pltpu.make_async_copy(src_hbm.at[pl.ds(0, BLK)], src_v.at[0], gi).wait()
            pltpu.make_async_copy(dst_hbm.at[pl.ds(0, BLK)], dst_v.at[0], gi).wait()

        prefetch_blk(0)

        pltpu.sync_copy(table.at[c, pl.ds(base, rows_per_tile)],
                        tab_sh.at[pl.ds(base, rows_per_tile)])

        zv = jnp.zeros((16,), jnp.float32)

        @pl.loop(0, CHUNK * (DH // 16))
        def _(t):
            i = t // (DH // 16)
            k = t % (DH // 16)
            buf_a[i, pl.ds(k * 16, 16)] = zv

        @pl.loop(0, CHUNK)
        def _(i):
            z16[i, pl.ds(0, 16)] = zv
            ones_v[i, pl.ds(0, 16)] = zv + 1.0

        for r in range(rows_per_tile // CHUNK):
            pltpu.sync_copy(buf_a, agg_sh.at[pl.ds(base + r * CHUNK, CHUNK)])
            if with_deg:
                pltpu.sync_copy(z16, deg_sh.at[pl.ds(base + r * CHUNK, CHUNK)])
        plsc.subcore_barrier()

        def src_row(j):
            return src_v.at[(j // BLK) % 3, j % BLK]

        def dst_row(j):
            return dst_v.at[(j // BLK) % 3, j % BLK]

        def issue_g(j, buf, sem):
            pltpu.async_copy(tab_sh.at[src_row(j)], buf, sem)

        def wait_g(buf, sem):
            pltpu.make_async_copy(tab_sh.at[src_row(0)], buf, sem).wait()

        def issue_s(j, buf, sem, parity):
            pltpu.async_copy(buf, agg_sh.at[dst_row(j)], sem, add=True)

        def drain_ones():
            pltpu.make_async_copy(deg_out.at[c, pl.ds(0, CHUNK)], ones_v,
                                  so).wait()

        def issue_ones(j, parity, drain):
            if not with_deg:
                return
            @pl.when(c == parity)
            def _():
                if drain:
                    drain_ones()
                pltpu.async_copy(ones_v, deg_sh.at[dst_row(j)], so, add=True)

        def wait_s(buf, sem):
            pltpu.make_async_copy(table.at[c, pl.ds(0, CHUNK)], buf, sem).wait()

        def crossing(j):
            @pl.when(j % BLK == BLK - 1)
            def _():
                wait_blk()

                @pl.when(j < nch - 2 * BLK)
                def _():
                    prefetch_blk((j + 1) // BLK + 1)

        wait_blk()
        prefetch_blk(1)
        issue_g(0, buf_a, ga)
        issue_g(1, buf_b, gb)
        wait_g(buf_a, ga)
        issue_s(0, buf_a, sa, 0)
        issue_ones(0, 0, drain=False)

        @pl.loop(0, (nch - 2) // 2)
        def _(jj):
            j1 = 2 * jj + 1
            crossing(j1)
            wait_s(buf_a, sa)
            issue_g(j1 + 1, buf_a, ga)
            wait_g(buf_b, gb)
            issue_s(j1, buf_b, sb, 1)
            if with_deg:
                @pl.when(jnp.logical_and(c == 1, jj > 0))
                def _():
                    drain_ones()
                @pl.when(c == 1)
                def _():
                    pltpu.async_copy(ones_v, deg_sh.at[dst_row(j1)], so, add=True)
            wait_s(buf_b, sb)
            issue_g(j1 + 2, buf_b, gb)
            wait_g(buf_a, ga)
            issue_s(j1 + 1, buf_a, sa, 0)
            issue_ones(j1 + 1, 0, drain=True)

        wait_s(buf_a, sa)
        wait_g(buf_b, gb)
        issue_s(nch - 1, buf_b, sb, 1)
        issue_ones(nch - 1, 1, drain=True)
        wait_s(buf_b, sb)
        if with_deg:
            drain_ones()

        plsc.subcore_barrier()

        pltpu.sync_copy(agg_sh.at[pl.ds(base, rows_per_tile)],
                        agg_out.at[c, pl.ds(base, rows_per_tile)])
        if with_deg:
            pltpu.sync_copy(deg_sh.at[pl.ds(base, rows_per_tile)],
                            deg_out.at[c, pl.ds(base, rows_per_tile)])

    return pl.kernel(
        body, out_type=out_type, mesh=mesh, scratch_types=scratch_types,
        compiler_params=pltpu.CompilerParams(use_tc_tiling_on_sc=False),
    )


def _pre1(x2, w_self, b, w_res):
    n_rows = x2.shape[1]
    blk = 1024

    def body(x_ref, ws_ref, b_ref, wr_ref, t_ref, r_ref):
        x = jnp.concatenate([x_ref[0], x_ref[1]], axis=1)
        t_ref[...] = jnp.dot(x, ws_ref[...], preferred_element_type=jnp.float32) + b_ref[...]
        r_ref[...] = jnp.dot(x, wr_ref[...], preferred_element_type=jnp.float32)

    return pl.pallas_call(
        body,
        grid=(n_rows // blk,),
        in_specs=[
            pl.BlockSpec((NC, blk, DH), lambda i: (0, i, 0)),
            pl.BlockSpec((D, D), lambda i: (0, 0)),
            pl.BlockSpec((1, D), lambda i: (0, 0)),
            pl.BlockSpec((D, D), lambda i: (0, 0)),
        ],
        out_specs=[
            pl.BlockSpec((blk, D), lambda i: (i, 0)),
            pl.BlockSpec((blk, D), lambda i: (i, 0)),
        ],
        out_shape=[
            jax.ShapeDtypeStruct((n_rows, D), jnp.float32),
            jax.ShapeDtypeStruct((n_rows, D), jnp.float32),
        ],
    )(x2, w_self, b.reshape(1, D), w_res)


def _post1(t1, aggp, deg, w_neigh):
    n_rows = t1.shape[0]
    blk = 1024

    def body(t_ref, a_ref, d_ref, wn_ref, o_ref):
        agg = jnp.concatenate([a_ref[0], a_ref[1]], axis=1)
        dg = d_ref[0, :, 0:1] + d_ref[1, :, 0:1]
        mean = agg / jnp.maximum(dg, 1.0)
        h = t_ref[...] + jnp.dot(mean, wn_ref[...], preferred_element_type=jnp.float32)
        h = jnp.maximum(h, 0.0)
        o_ref[0] = h[:, :DH]
        o_ref[1] = h[:, DH:]

    return pl.pallas_call(
        body,
        grid=(n_rows // blk,),
        in_specs=[
            pl.BlockSpec((blk, D), lambda i: (i, 0)),
            pl.BlockSpec((NC, blk, DH), lambda i: (0, i, 0)),
            pl.BlockSpec((NC, blk, 16), lambda i: (0, i, 0)),
            pl.BlockSpec((D, D), lambda i: (0, 0)),
        ],
        out_specs=pl.BlockSpec((NC, blk, DH), lambda i: (0, i, 0)),
        out_shape=jax.ShapeDtypeStruct((NC, n_rows, DH), jnp.float32),
    )(t1, aggp, deg, w_neigh)


def _pre2(h2, r, w_self, b):
    n_rows = h2.shape[1]
    blk = 1024

    def body(h_ref, r_ref, ws_ref, b_ref, o_ref):
        h1 = jnp.concatenate([h_ref[0], h_ref[1]], axis=1)
        o_ref[...] = (jnp.dot(h1, ws_ref[...], preferred_element_type=jnp.float32)
                      + b_ref[...] + r_ref[...])

    return pl.pallas_call(
        body,
        grid=(n_rows // blk,),
        in_specs=[
            pl.BlockSpec((NC, blk, DH), lambda i: (0, i, 0)),
            pl.BlockSpec((blk, D), lambda i: (i, 0)),
            pl.BlockSpec((D, D), lambda i: (0, 0)),
            pl.BlockSpec((1, D), lambda i: (0, 0)),
        ],
        out_specs=pl.BlockSpec((blk, D), lambda i: (i, 0)),
        out_shape=jax.ShapeDtypeStruct((n_rows, D), jnp.float32),
    )(h2, r, w_self, b.reshape(1, D))


def _post2(t2, aggp, deg, w_neigh, w_mlp_pad, b_mlp_pad):
    n_rows = t2.shape[0]
    blk = 1024

    def body(t_ref, a_ref, d_ref, wn_ref, wm_ref, bm_ref, o_ref):
        agg = jnp.concatenate([a_ref[0], a_ref[1]], axis=1)
        dg = d_ref[0, :, 0:1] + d_ref[1, :, 0:1]
        mean = agg / jnp.maximum(dg, 1.0)
        h = t_ref[...] + jnp.dot(mean, wn_ref[...], preferred_element_type=jnp.float32)
        o_ref[...] = jnp.dot(h, wm_ref[...], preferred_element_type=jnp.float32) + bm_ref[...]

    return pl.pallas_call(
        body,
        grid=(n_rows // blk,),
        in_specs=[
            pl.BlockSpec((blk, D), lambda i: (i, 0)),
            pl.BlockSpec((NC, blk, DH), lambda i: (0, i, 0)),
            pl.BlockSpec((NC, blk, 16), lambda i: (0, i, 0)),
            pl.BlockSpec((D, D), lambda i: (0, 0)),
            pl.BlockSpec((D, D), lambda i: (0, 0)),
            pl.BlockSpec((1, D), lambda i: (0, 0)),
        ],
        out_specs=pl.BlockSpec((blk, D), lambda i: (i, 0)),
        out_shape=jax.ShapeDtypeStruct((n_rows, D), jnp.float32),
    )(t2, aggp, deg, w_neigh, w_mlp_pad, b_mlp_pad)


def kernel(node_feats, edge_index, W_self1, W_neigh1, b1, W_self2, W_neigh2, b2, W_res, W_mlp, b_mlp):
    n = node_feats.shape[0]
    e = edge_index.shape[1]
    src = edge_index[0]
    dst = edge_index[1]

    rows_per_tile = -(-(n + 1) // (NS * CHUNK)) * CHUNK
    n_rows = rows_per_tile * NS

    nch = -(-e // (NS * CHUNK))
    nch = -(-nch // 16) * 16
    e_pad = NS * nch * CHUNK
    pad_i = jnp.arange(e_pad - e, dtype=jnp.int32)
    src_p = jnp.concatenate([src, pad_i % n])
    dst_p = jnp.concatenate([dst, n + pad_i % (n_rows - n)])
    src_p = src_p.reshape(e_pad // CHUNK, CHUNK)
    dst_p = dst_p.reshape(e_pad // CHUNK, CHUNK)

    x_pad = jnp.zeros((n_rows, D), jnp.float32).at[:n].set(node_feats)
    x2 = jnp.stack([x_pad[:, :DH], x_pad[:, DH:]])

    seg1 = _make_seg_kernel(nch, n_rows, rows_per_tile, with_deg=True)
    aggp1, deg = seg1(x2, src_p, dst_p)
    t1, r = _pre1(x2, W_self1, b1, W_res)

    h2 = _post1(t1, aggp1, deg, W_neigh1)

    seg2 = _make_seg_kernel(nch, n_rows, rows_per_tile, with_deg=False)
    (aggp2,) = seg2(h2, src_p, dst_p)
    t2 = _pre2(h2, r, W_self2, b2)

    w_mlp_pad = jnp.zeros((D, D), jnp.float32).at[:, : W_mlp.shape[1]].set(W_mlp)
    b_mlp_pad = jnp.zeros((1, D), jnp.float32).at[0, : W_mlp.shape[1]].set(b_mlp)

    out = _post2(t2, aggp2, deg, W_neigh2, w_mlp_pad, b_mlp_pad)
    return out[:n, : W_mlp.shape[1]]


# speedup vs baseline: 1.6976x; 1.0075x over previous
import jax
import jax.numpy as jnp
from jax import lax
from jax.experimental import pallas as pl
from jax.experimental.pallas import tpu as pltpu
from jax.experimental.pallas import tpu_sc as plsc

D = 128
DH = 64
NC = 2
NS = 16
CHUNK = 128


def _make_seg_kernel(nch, n_rows, rows_per_tile, with_deg):
    mesh = plsc.VectorSubcoreMesh(
        core_axis_name="c", subcore_axis_name="s", num_cores=NC, num_subcores=NS
    )
    BLK = 16
    NBLK = nch // BLK
    assert nch % BLK == 0 and nch % 2 == 0 and NBLK >= 4
    out_type = [jax.ShapeDtypeStruct((NC, n_rows, DH), jnp.float32)]
    if with_deg:
        out_type.append(jax.ShapeDtypeStruct((NC, n_rows, 16), jnp.float32))
    scratch_types = [
        pltpu.VMEM((3, BLK, CHUNK), jnp.int32),
        pltpu.VMEM((3, BLK, CHUNK), jnp.int32),
        pltpu.VMEM((CHUNK, DH), jnp.float32),
        pltpu.VMEM((CHUNK, DH), jnp.float32),
        pltpu.VMEM((CHUNK, 16), jnp.float32),
        pltpu.VMEM((CHUNK, 16), jnp.float32),
        pltpu.VMEM_SHARED((n_rows, DH), jnp.float32),
        pltpu.VMEM_SHARED((n_rows, DH), jnp.float32),
        pltpu.VMEM_SHARED((n_rows, 16), jnp.float32),
        pltpu.SemaphoreType.DMA,
        pltpu.SemaphoreType.DMA,
        pltpu.SemaphoreType.DMA,
        pltpu.SemaphoreType.DMA,
        pltpu.SemaphoreType.DMA,
        pltpu.SemaphoreType.DMA,
    ]

    def body(table, src_hbm, dst_hbm, *rest):
        if with_deg:
            (agg_out, deg_out, src_v, dst_v, buf_a, buf_b, ones_v, z16,
             tab_sh, agg_sh, deg_sh, gi, ga, gb, sa, sb, so) = rest
        else:
            (agg_out, src_v, dst_v, buf_a, buf_b, ones_v, z16,
             tab_sh, agg_sh, deg_sh, gi, ga, gb, sa, sb, so) = rest
        c = lax.axis_index("c")
        s = lax.axis_index("s")
        base = s * rows_per_tile

        def prefetch_blk(b):
            pltpu.async_copy(src_hbm.at[pl.ds(s * nch + b * BLK, BLK)],
                             src_v.at[b % 3], gi)
            pltpu.async_copy(dst_hbm.at[pl.ds(s * nch + b * BLK, BLK)],
                             dst_v.at[b % 3], gi)

        def wait_blk():
            pltpu.make_async_copy(src_hbm.at[pl.ds(0, BLK)], src_v.at[0], gi).wait()
            pltpu.make_async_copy(dst_hbm.at[pl.ds(0, BLK)], dst_v.at[0], gi).wait()

        prefetch_blk(0)

        pltpu.sync_copy(table.at[c, pl.ds(base, rows_per_tile)],
                        tab_sh.at[pl.ds(base, rows_per_tile)])

        zv = jnp.zeros((16,), jnp.float32)

        @pl.loop(0, CHUNK * (DH // 16))
        def _(t):
            i = t // (DH // 16)
            k = t % (DH // 16)
            buf_a[i, pl.ds(k * 16, 16)] = zv

        @pl.loop(0, CHUNK)
        def _(i):
            z16[i, pl.ds(0, 16)] = zv
            ones_v[i, pl.ds(0, 16)] = zv + 1.0

        for r in range(rows_per_tile // CHUNK):
            pltpu.sync_copy(buf_a, agg_sh.at[pl.ds(base + r * CHUNK, CHUNK)])
            if with_deg:
                pltpu.sync_copy(z16, deg_sh.at[pl.ds(base + r * CHUNK, CHUNK)])
        plsc.subcore_barrier()

        def src_row(j):
            return src_v.at[(j // BLK) % 3, j % BLK]

        def dst_row(j):
            return dst_v.at[(j // BLK) % 3, j % BLK]

        def issue_g(j, buf, sem):
            pltpu.async_copy(tab_sh.at[src_row(j)], buf, sem)

        def wait_g(buf, sem):
            pltpu.make_async_copy(tab_sh.at[src_row(0)], buf, sem).wait()

        def issue_s(j, buf, sem, parity):
            pltpu.async_copy(buf, agg_sh.at[dst_row(j)], sem, add=True)

        def drain_ones():
            pltpu.make_async_copy(deg_out.at[c, pl.ds(0, CHUNK)], ones_v,
                                  so).wait()

        def issue_ones(j, parity, drain):
            if not with_deg:
                return
            @pl.when(c == parity)
            def _():
                if drain:
                    drain_ones()
                pltpu.async_copy(ones_v, deg_sh.at[dst_row(j)], so, add=True)

        def wait_s(buf, sem):
            pltpu.make_async_copy(table.at[c, pl.ds(0, CHUNK)], buf, sem).wait()

        def crossing(j):
            @pl.when(j % BLK == BLK - 1)
            def _():
                wait_blk()

                @pl.when(j < nch - 2 * BLK)
                def _():
                    prefetch_blk((j + 1) // BLK + 1)

        wait_blk()
        prefetch_blk(1)
        issue_g(0, buf_a, ga)
        issue_g(1, buf_b, gb)
        wait_g(buf_a, ga)
        issue_s(0, buf_a, sa, 0)
        issue_ones(0, 0, drain=False)

        @pl.loop(0, (nch - 2) // 2)
        def _(jj):
            j1 = 2 * jj + 1
            crossing(j1)
            wait_s(buf_a, sa)
            issue_g(j1 + 1, buf_a, ga)
            wait_g(buf_b, gb)
            issue_s(j1, buf_b, sb, 1)
            if with_deg:
                @pl.when(jnp.logical_and(c == 1, jj > 0))
                def _():
                    drain_ones()
                @pl.when(c == 1)
                def _():
                    pltpu.async_copy(ones_v, deg_sh.at[dst_row(j1)], so, add=True)
            wait_s(buf_b, sb)
            issue_g(j1 + 2, buf_b, gb)
            wait_g(buf_a, ga)
            issue_s(j1 + 1, buf_a, sa, 0)
            issue_ones(j1 + 1, 0, drain=True)

        wait_s(buf_a, sa)
        wait_g(buf_b, gb)
        issue_s(nch - 1, buf_b, sb, 1)
        issue_ones(nch - 1, 1, drain=True)
        wait_s(buf_b, sb)
        if with_deg:
            drain_ones()

        plsc.subcore_barrier()

        pltpu.sync_copy(agg_sh.at[pl.ds(base, rows_per_tile)],
                        agg_out.at[c, pl.ds(base, rows_per_tile)])
        if with_deg:
            pltpu.sync_copy(deg_sh.at[pl.ds(base, rows_per_tile)],
                            deg_out.at[c, pl.ds(base, rows_per_tile)])

    return pl.kernel(
        body, out_type=out_type, mesh=mesh, scratch_types=scratch_types,
        compiler_params=pltpu.CompilerParams(use_tc_tiling_on_sc=False),
    )


def _pre1(x2, w_self, b, w_res):
    n_rows = x2.shape[1]
    blk = 2048

    def body(x_ref, ws_ref, b_ref, wr_ref, t_ref, r_ref):
        x = jnp.concatenate([x_ref[0], x_ref[1]], axis=1)
        t_ref[...] = jnp.dot(x, ws_ref[...], preferred_element_type=jnp.float32) + b_ref[...]
        r_ref[...] = jnp.dot(x, wr_ref[...], preferred_element_type=jnp.float32)

    return pl.pallas_call(
        body,
        grid=(n_rows // blk,),
        in_specs=[
            pl.BlockSpec((NC, blk, DH), lambda i: (0, i, 0)),
            pl.BlockSpec((D, D), lambda i: (0, 0)),
            pl.BlockSpec((1, D), lambda i: (0, 0)),
            pl.BlockSpec((D, D), lambda i: (0, 0)),
        ],
        out_specs=[
            pl.BlockSpec((blk, D), lambda i: (i, 0)),
            pl.BlockSpec((blk, D), lambda i: (i, 0)),
        ],
        out_shape=[
            jax.ShapeDtypeStruct((n_rows, D), jnp.float32),
            jax.ShapeDtypeStruct((n_rows, D), jnp.float32),
        ],
    )(x2, w_self, b.reshape(1, D), w_res)


def _post1(t1, aggp, deg, w_neigh):
    n_rows = t1.shape[0]
    blk = 2048

    def body(t_ref, a_ref, d_ref, wn_ref, o_ref):
        agg = jnp.concatenate([a_ref[0], a_ref[1]], axis=1)
        dg = d_ref[0, :, 0:1] + d_ref[1, :, 0:1]
        mean = agg / jnp.maximum(dg, 1.0)
        h = t_ref[...] + jnp.dot(mean, wn_ref[...], preferred_element_type=jnp.float32)
        h = jnp.maximum(h, 0.0)
        o_ref[0] = h[:, :DH]
        o_ref[1] = h[:, DH:]

    return pl.pallas_call(
        body,
        grid=(n_rows // blk,),
        in_specs=[
            pl.BlockSpec((blk, D), lambda i: (i, 0)),
            pl.BlockSpec((NC, blk, DH), lambda i: (0, i, 0)),
            pl.BlockSpec((NC, blk, 16), lambda i: (0, i, 0)),
            pl.BlockSpec((D, D), lambda i: (0, 0)),
        ],
        out_specs=pl.BlockSpec((NC, blk, DH), lambda i: (0, i, 0)),
        out_shape=jax.ShapeDtypeStruct((NC, n_rows, DH), jnp.float32),
    )(t1, aggp, deg, w_neigh)


def _pre2(h2, r, w_self, b):
    n_rows = h2.shape[1]
    blk = 2048

    def body(h_ref, r_ref, ws_ref, b_ref, o_ref):
        h1 = jnp.concatenate([h_ref[0], h_ref[1]], axis=1)
        o_ref[...] = (jnp.dot(h1, ws_ref[...], preferred_element_type=jnp.float32)
                      + b_ref[...] + r_ref[...])

    return pl.pallas_call(
        body,
        grid=(n_rows // blk,),
        in_specs=[
            pl.BlockSpec((NC, blk, DH), lambda i: (0, i, 0)),
            pl.BlockSpec((blk, D), lambda i: (i, 0)),
            pl.BlockSpec((D, D), lambda i: (0, 0)),
            pl.BlockSpec((1, D), lambda i: (0, 0)),
        ],
        out_specs=pl.BlockSpec((blk, D), lambda i: (i, 0)),
        out_shape=jax.ShapeDtypeStruct((n_rows, D), jnp.float32),
    )(h2, r, w_self, b.reshape(1, D))


def _post2(t2, aggp, deg, w_neigh, w_mlp_pad, b_mlp_pad):
    n_rows = t2.shape[0]
    blk = 2048

    def body(t_ref, a_ref, d_ref, wn_ref, wm_ref, bm_ref, o_ref):
        agg = jnp.concatenate([a_ref[0], a_ref[1]], axis=1)
        dg = d_ref[0, :, 0:1] + d_ref[1, :, 0:1]
        mean = agg / jnp.maximum(dg, 1.0)
        h = t_ref[...] + jnp.dot(mean, wn_ref[...], preferred_element_type=jnp.float32)
        o_ref[...] = jnp.dot(h, wm_ref[...], preferred_element_type=jnp.float32) + bm_ref[...]

    return pl.pallas_call(
        body,
        grid=(n_rows // blk,),
        in_specs=[
            pl.BlockSpec((blk, D), lambda i: (i, 0)),
            pl.BlockSpec((NC, blk, DH), lambda i: (0, i, 0)),
            pl.BlockSpec((NC, blk, 16), lambda i: (0, i, 0)),
            pl.BlockSpec((D, D), lambda i: (0, 0)),
            pl.BlockSpec((D, D), lambda i: (0, 0)),
            pl.BlockSpec((1, D), lambda i: (0, 0)),
        ],
        out_specs=pl.BlockSpec((blk, D), lambda i: (i, 0)),
        out_shape=jax.ShapeDtypeStruct((n_rows, D), jnp.float32),
    )(t2, aggp, deg, w_neigh, w_mlp_pad, b_mlp_pad)


def kernel(node_feats, edge_index, W_self1, W_neigh1, b1, W_self2, W_neigh2, b2, W_res, W_mlp, b_mlp):
    n = node_feats.shape[0]
    e = edge_index.shape[1]
    src = edge_index[0]
    dst = edge_index[1]

    rows_per_tile = -(-(n + 1) // (NS * CHUNK)) * CHUNK
    n_rows = rows_per_tile * NS

    nch = -(-e // (NS * CHUNK))
    nch = -(-nch // 16) * 16
    e_pad = NS * nch * CHUNK
    pad_i = jnp.arange(e_pad - e, dtype=jnp.int32)
    src_p = jnp.concatenate([src, pad_i % n])
    dst_p = jnp.concatenate([dst, n + pad_i % (n_rows - n)])
    src_p = src_p.reshape(e_pad // CHUNK, CHUNK)
    dst_p = dst_p.reshape(e_pad // CHUNK, CHUNK)

    x_pad = jnp.zeros((n_rows, D), jnp.float32).at[:n].set(node_feats)
    x2 = jnp.stack([x_pad[:, :DH], x_pad[:, DH:]])

    seg1 = _make_seg_kernel(nch, n_rows, rows_per_tile, with_deg=True)
    aggp1, deg = seg1(x2, src_p, dst_p)
    t1, r = _pre1(x2, W_self1, b1, W_res)

    h2 = _post1(t1, aggp1, deg, W_neigh1)

    seg2 = _make_seg_kernel(nch, n_rows, rows_per_tile, with_deg=False)
    (aggp2,) = seg2(h2, src_p, dst_p)
    t2 = _pre2(h2, r, W_self2, b2)

    w_mlp_pad = jnp.zeros((D, D), jnp.float32).at[:, : W_mlp.shape[1]].set(W_mlp)
    b_mlp_pad = jnp.zeros((1, D), jnp.float32).at[0, : W_mlp.shape[1]].set(b_mlp)

    out = _post2(t2, aggp2, deg, W_neigh2, w_mlp_pad, b_mlp_pad)
    return out[:n, : W_mlp.shape[1]]
